# baseline (speedup 1.0000x reference)
import jax
import jax.numpy as jnp
import numpy as np
from jax.experimental import pallas as pl

EXTENT = float(np.float32(1.5 * 6 * 0.025))
RADIUS = EXTENT / 2.0


def _sgn(v):
    return jnp.where(v >= 0, 1.0, -1.0)


def _ball_to_cube(p):
    eps = 1e-8
    x, y, z = p[:, 0], p[:, 1], p[:, 2]
    sq = x * x + y * y + z * z
    norm = jnp.sqrt(sq + eps)
    condA = (5.0 / 4.0) * z * z > (x * x + y * y)
    sA = jnp.sqrt(3.0 * norm / (norm + jnp.abs(z) + eps))
    rxy = jnp.sqrt(x * x + y * y + eps)
    sB = norm / rxy
    cx = jnp.where(condA, x * sA, x * sB)
    cy = jnp.where(condA, y * sA, y * sB)
    cz = jnp.where(condA, _sgn(z) * norm, 1.5 * z)
    is0 = sq < 1e-12
    cx = jnp.where(is0, 0.0, cx)
    cy = jnp.where(is0, 0.0, cy)
    cz = jnp.where(is0, 0.0, cz)
    sq_xy = cx * cx + cy * cy
    r = jnp.sqrt(sq_xy + eps)
    condC = jnp.abs(cy) <= jnp.abs(cx)
    denx = jnp.where(jnp.abs(cx) > eps, cx, 1.0)
    deny = jnp.where(jnp.abs(cy) > eps, cy, 1.0)
    aC = _sgn(cx) * r
    bC = aC * (4.0 / jnp.pi) * jnp.arctan(cy / denx)
    bD = _sgn(cy) * r
    aD = bD * (4.0 / jnp.pi) * jnp.arctan(cx / deny)
    ux = jnp.where(condC, aC, aD)
    uy = jnp.where(condC, bC, bD)
    xy0 = sq_xy < 1e-12
    ux = jnp.where(xy0, 0.0, ux)
    uy = jnp.where(xy0, 0.0, uy)
    return jnp.stack([ux, uy, cz], axis=1)


def _cconv(feat, pos, edge_src, edge_dst, W):
    n = pos.shape[0]
    in_ch = feat.shape[1]
    kd, kh, kw = W.shape[0], W.shape[1], W.shape[2]
    kk = kd * kh * kw
    out_ch = W.shape[-1]
    rel = (pos[edge_src] - pos[edge_dst]) / RADIUS
    r_sqr = jnp.sum(rel * rel, axis=1)
    win = jnp.clip((1.0 - r_sqr) ** 3, 0.0, 1.0)
    mapped = jnp.clip(_ball_to_cube(rel), -1.0, 1.0)
    tx = (mapped[:, 0] + 1.0) * 0.5 * (kw - 1)
    ty = (mapped[:, 1] + 1.0) * 0.5 * (kh - 1)
    tz = (mapped[:, 2] + 1.0) * 0.5 * (kd - 1)
    fx = jnp.clip(jnp.floor(tx), 0, kw - 2)
    fy = jnp.clip(jnp.floor(ty), 0, kh - 2)
    fz = jnp.clip(jnp.floor(tz), 0, kd - 2)
    wx1, wy1, wz1 = tx - fx, ty - fy, tz - fz
    ix = fx.astype(jnp.int32)
    iy = fy.astype(jnp.int32)
    iz = fz.astype(jnp.int32)
    base = feat[edge_src] * win[:, None]
    acc = jnp.zeros((n * kk, in_ch), feat.dtype)
    for dz in (0, 1):
        wz = wz1 if dz else (1.0 - wz1)
        for dy in (0, 1):
            wy = wy1 if dy else (1.0 - wy1)
            for dx in (0, 1):
                wxx = wx1 if dx else (1.0 - wx1)
                w = (wz * wy * wxx)[:, None]
                cell = ((iz + dz) * kh + (iy + dy)) * kw + (ix + dx)
                acc = acc.at[edge_dst * kk + cell].add(base * w)
    acc = acc.reshape(n, kk, in_ch)
    return jnp.einsum('nki,kio->no', acc, W.reshape(kk, in_ch, out_ch))


def kernel(xyz, feats, conv0_w, dense0_w, dense0_b, conv1_w, dense1_w, dense1_b,
           conv2_w, dense2_w, dense2_b, conv3_w, dense3_w, dense3_b,
           edge_src, edge_dst):
    f = jnp.concatenate([jnp.ones_like(xyz[:, 0:1]), feats], axis=-1)
    c0 = _cconv(f, xyz, edge_src, edge_dst, conv0_w)
    d0 = f @ dense0_w.T + dense0_b
    h = jnp.concatenate([c0, d0], axis=-1)
    layers = ((conv1_w, dense1_w, dense1_b),
              (conv2_w, dense2_w, dense2_b),
              (conv3_w, dense3_w, dense3_b))
    for cw, dw, db in layers:
        inp = jax.nn.relu(h)
        c = _cconv(inp, xyz, edge_src, edge_dst, cw)
        d = inp @ dw.T + db
        if d.shape[-1] == h.shape[-1]:
            h = c + d + h
        else:
            h = c + d
    return (1.0 / 128.0) * h


# speedup vs baseline: 6.6198x; 6.6198x over previous
import functools

import jax
import jax.numpy as jnp
import numpy as np
from jax import lax
from jax.experimental import pallas as pl
from jax.experimental.pallas import tpu as pltpu
from jax.experimental.pallas import tpu_sc as plsc

EXTENT = float(np.float32(1.5 * 6 * 0.025))
RADIUS = EXTENT / 2.0

_NC = 2
_NS = 16
_NW = _NC * _NS

_G = 16
_ROWS = _G * 64
_SB = 256
_GEO_B = 1024

_OFF = (0, 1, 4, 5, 16, 17, 20, 21)

_ATAN_C = (0.9999903820069088, -0.333001703445317, 0.19663803791358045,
           -0.12725099600657397, 0.07100970338731588, -0.026659183461222813,
           0.004672589475411147)


def _v_sqrt(v):
    i = lax.bitcast_convert_type(v, jnp.int32)
    y = lax.bitcast_convert_type(jnp.int32(0x5F3759DF) - (i >> 1), jnp.float32)
    for _ in range(3):
        y = y * (1.5 - 0.5 * v * y * y)
    return v * y


def _v_atan(x):
    x = jnp.clip(x, -1.1, 1.1)
    t = x * x
    p = jnp.float32(_ATAN_C[6])
    for k in range(5, -1, -1):
        p = p * t + jnp.float32(_ATAN_C[k])
    return p * x


def _sgn(v):
    return jnp.where(v >= 0.0, 1.0, -1.0)


def _edge_geometry(xs, ys, zs, xd, yd, zd):
    inv_r = jnp.float32(1.0 / RADIUS)
    x = (xs - xd) * inv_r
    y = (ys - yd) * inv_r
    z = (zs - zd) * inv_r
    eps = jnp.float32(1e-8)
    sq = x * x + y * y + z * z
    norm = _v_sqrt(sq + eps)
    condA = (5.0 / 4.0) * z * z > (x * x + y * y)
    sA = _v_sqrt(3.0 * norm / (norm + jnp.abs(z) + eps))
    rxy = _v_sqrt(x * x + y * y + eps)
    sB = norm / rxy
    cx = jnp.where(condA, x * sA, x * sB)
    cy = jnp.where(condA, y * sA, y * sB)
    cz = jnp.where(condA, _sgn(z) * norm, 1.5 * z)
    is0 = sq < 1e-12
    zero = jnp.zeros_like(cx)
    cx = jnp.where(is0, zero, cx)
    cy = jnp.where(is0, zero, cy)
    cz = jnp.where(is0, zero, cz)
    sq_xy = cx * cx + cy * cy
    r = _v_sqrt(sq_xy + eps)
    condC = jnp.abs(cy) <= jnp.abs(cx)
    denx = jnp.where(jnp.abs(cx) > eps, cx, jnp.float32(1.0))
    deny = jnp.where(jnp.abs(cy) > eps, cy, jnp.float32(1.0))
    aC = _sgn(cx) * r
    bC = aC * jnp.float32(4.0 / np.pi) * _v_atan(cy / denx)
    bD = _sgn(cy) * r
    aD = bD * jnp.float32(4.0 / np.pi) * _v_atan(cx / deny)
    ux = jnp.where(condC, aC, aD)
    uy = jnp.where(condC, bC, bD)
    xy0 = sq_xy < 1e-12
    ux = jnp.where(xy0, zero, ux)
    uy = jnp.where(xy0, zero, uy)

    one = jnp.float32(1.0)
    omsq = one - sq
    win = jnp.clip(omsq * omsq * omsq, 0.0, 1.0)
    mx = jnp.clip(ux, -1.0, 1.0)
    my = jnp.clip(uy, -1.0, 1.0)
    mz = jnp.clip(cz, -1.0, 1.0)
    tx = (mx + one) * 1.5
    ty = (my + one) * 1.5
    tz = (mz + one) * 1.5
    ix = jnp.minimum(tx.astype(jnp.int32), 2)
    iy = jnp.minimum(ty.astype(jnp.int32), 2)
    iz = jnp.minimum(tz.astype(jnp.int32), 2)
    wx1 = tx - ix.astype(jnp.float32)
    wy1 = ty - iy.astype(jnp.float32)
    wz1 = tz - iz.astype(jnp.float32)
    wx0 = one - wx1
    wy0 = one - wy1
    wz0 = one - wz1
    ws = []
    for wz in (wz0, wz1):
        for wy in (wy0, wy1):
            wyz = win * wz * wy
            for wx in (wx0, wx1):
                ws.append(wyz * wx)
    cellbase = (iz * 4 + iy) * 4 + ix
    return ws, cellbase


def _make_geometry_kernel(N, Ep):
    per_tile = Ep // _NW
    nbatch = (per_tile + _GEO_B - 1) // _GEO_B
    mesh = plsc.VectorSubcoreMesh(core_axis_name="c", subcore_axis_name="s")

    @functools.partial(
        pl.kernel,
        mesh=mesh,
        compiler_params=pltpu.CompilerParams(needs_layout_passes=False),
        out_type=[
            jax.ShapeDtypeStruct((8, Ep), jnp.float32),
            jax.ShapeDtypeStruct((Ep,), jnp.int32),
        ],
        scratch_types=[
            pltpu.VMEM((N,), jnp.float32),
            pltpu.VMEM((N,), jnp.float32),
            pltpu.VMEM((N,), jnp.float32),
            pltpu.VMEM((_GEO_B,), jnp.int32),
            pltpu.VMEM((_GEO_B,), jnp.int32),
            pltpu.VMEM((8, _GEO_B), jnp.float32),
            pltpu.VMEM((_GEO_B,), jnp.int32),
        ],
    )
    def geom_kernel(xs_h, ys_h, zs_h, src_h, dst_h, wgt_h, rowb_h,
                    xs_v, ys_v, zs_v, sbuf, dbuf, wout, rout):
        wid = lax.axis_index("s") * _NC + lax.axis_index("c")
        base = wid * per_tile
        pltpu.sync_copy(xs_h, xs_v)
        pltpu.sync_copy(ys_h, ys_v)
        pltpu.sync_copy(zs_h, zs_v)
        iota = lax.iota(jnp.int32, 16)

        for b in range(nbatch):
            b0 = base + b * _GEO_B
            sb = min(_GEO_B, per_tile - b * _GEO_B)
            pltpu.sync_copy(src_h.at[pl.ds(b0, sb)], sbuf.at[pl.ds(0, sb)])
            pltpu.sync_copy(dst_h.at[pl.ds(b0, sb)], dbuf.at[pl.ds(0, sb)])

            def group(g, carry):
                col = g * 16 + iota
                sv = plsc.load_gather(sbuf, [col])
                dv = plsc.load_gather(dbuf, [col])
                xsv = plsc.load_gather(xs_v, [sv])
                ysv = plsc.load_gather(ys_v, [sv])
                zsv = plsc.load_gather(zs_v, [sv])
                xdv = plsc.load_gather(xs_v, [dv])
                ydv = plsc.load_gather(ys_v, [dv])
                zdv = plsc.load_gather(zs_v, [dv])
                ws, cellbase = _edge_geometry(xsv, ysv, zsv, xdv, ydv, zdv)
                for k in range(8):
                    plsc.store_scatter(
                        wout, [jnp.full((16,), k, jnp.int32), col], ws[k])
                plsc.store_scatter(rout, [col], dv * 64 + cellbase)
                return carry

            lax.fori_loop(0, sb // 16, group, jnp.int32(0))
            pltpu.sync_copy(wout.at[:, pl.ds(0, sb)],
                            wgt_h.at[:, pl.ds(b0, sb)])
            pltpu.sync_copy(rout.at[pl.ds(0, sb)], rowb_h.at[pl.ds(b0, sb)])

    return geom_kernel


def _make_scatter_kernel(Np, Ep, C, in_p):
    nq = in_p // 16
    chunks_per_tile = C // _NW
    acc_words = _ROWS * in_p
    mesh = plsc.VectorSubcoreMesh(core_axis_name="c", subcore_axis_name="s")

    @functools.partial(
        pl.kernel,
        mesh=mesh,
        compiler_params=pltpu.CompilerParams(needs_layout_passes=False,
                                             use_tc_tiling_on_sc=False),
        out_type=jax.ShapeDtypeStruct((Np * 64 * in_p,), jnp.float32),
        scratch_types=[
            pltpu.VMEM((acc_words,), jnp.float32),
            pltpu.VMEM((_SB, in_p), jnp.float32),
            pltpu.VMEM((_SB // 2,), jnp.int32),
            pltpu.VMEM((_SB // 2,), jnp.int32),
            pltpu.VMEM((8, _SB + 16), jnp.float32),
            pltpu.VMEM((_SB + 16,), jnp.int32),
            pltpu.VMEM((C + 1 + 16,), jnp.int32),
            pltpu.SemaphoreType.DMA,
        ],
    )
    def scatter_kernel(feat_h, wgt_h, rowb_h, offs_h, src_h, acc_h,
                       accv, rows_v, sbuf0, sbuf1, wbuf, rbuf, offs_v, sem):
        wid = lax.axis_index("s") * _NC + lax.axis_index("c")
        pltpu.sync_copy(offs_h, offs_v.at[pl.ds(0, C + 1)])
        iota = lax.iota(jnp.int32, 16)
        zeros16 = jnp.zeros((16,), jnp.float32)
        cols = [q * 16 + iota for q in range(nq)]

        def chunk_body(t, carry):
            c = wid + t * _NW
            ov = offs_v[pl.ds(c, 16)]
            e0 = ov[0]
            e1 = ov[1]
            row0 = c * _ROWS

            def zbody(i, carry2):
                b = i * 64
                for o in range(4):
                    plsc.store_scatter(accv, [b + o * 16 + iota], zeros16)
                return carry2

            lax.fori_loop(0, acc_words // 64, zbody, jnp.int32(0))

            b0a = pl.multiple_of(e0 & jnp.int32(~127), 128)
            nb = (e1 - b0a + jnp.int32(_SB - 1)) // jnp.int32(_SB)

            def batch_body(tb, carry3):
                b0 = pl.multiple_of(b0a + tb * _SB, 128)
                pltpu.sync_copy(src_h.at[pl.ds(b0, _SB // 2)], sbuf0)
                pltpu.sync_copy(src_h.at[pl.ds(b0 + _SB // 2, _SB // 2)],
                                sbuf1)
                pltpu.sync_copy(wgt_h.at[:, pl.ds(b0, _SB)],
                                wbuf.at[:, pl.ds(0, _SB)])
                pltpu.sync_copy(rowb_h.at[pl.ds(b0, _SB)],
                                rbuf.at[pl.ds(0, _SB)])
                cp0 = pltpu.async_copy(
                    feat_h.at[sbuf0], rows_v.at[pl.ds(0, _SB // 2)], sem)
                cp1 = pltpu.async_copy(
                    feat_h.at[sbuf1], rows_v.at[pl.ds(_SB // 2, _SB // 2)],
                    sem)
                cp0.wait()
                cp1.wait()
                js = jnp.maximum(e0 - b0, 0)
                je = jnp.minimum(e1 - b0, jnp.int32(_SB))

                def edge_body(j, carry4):
                    jv = lax.broadcast(j, (16,))
                    rv = [plsc.load_gather(rows_v, [jv, cols[q]])
                          for q in range(nq)]
                    rowloc = rbuf[pl.ds(j, 16)][0] - row0

                    for k in range(8):
                        wv = lax.broadcast(wbuf[k, pl.ds(j, 16)][0], (16,))
                        rb = (rowloc + _OFF[k]) * in_p
                        for q in range(nq):
                            plsc.addupdate(
                                accv.at[pl.ds(rb + q * 16, 16)],
                                wv * rv[q])
                    return carry4

                lax.fori_loop(js, je, edge_body, jnp.int32(0))
                return carry3

            lax.fori_loop(0, nb, batch_body, jnp.int32(0))
            pltpu.sync_copy(
                accv,
                acc_h.at[pl.ds(pl.multiple_of(row0 * in_p, acc_words),
                               acc_words)])
            return carry

        lax.fori_loop(0, chunks_per_tile, chunk_body, jnp.int32(0))

    return scatter_kernel


def _tc_layer(acc2, wr, inp, dwT, db2, hres, want_relu, scale):
    Np, K = acc2.shape
    out = wr.shape[1]
    inch = inp.shape[1]
    BN = 512
    grid = (Np // BN,)

    def body(*refs):
        if hres is not None:
            acc_r, wr_r, inp_r, dw_r, db_r, hres_r = refs[:6]
            outs = refs[6:]
        else:
            acc_r, wr_r, inp_r, dw_r, db_r = refs[:5]
            outs = refs[5:]
        o = jnp.dot(acc_r[...], wr_r[...], preferred_element_type=jnp.float32)
        o = o + jnp.dot(inp_r[...], dw_r[...],
                        preferred_element_type=jnp.float32)
        o = o + db_r[...]
        if hres is not None:
            o = o + hres_r[...]
        if scale != 1.0:
            o = o * jnp.float32(scale)
        outs[0][...] = o
        if want_relu:
            outs[1][...] = jnp.maximum(o, 0.0)

    in_specs = [
        pl.BlockSpec((BN, K), lambda i: (i, 0)),
        pl.BlockSpec((K, out), lambda i: (0, 0)),
        pl.BlockSpec((BN, inch), lambda i: (i, 0)),
        pl.BlockSpec((inch, out), lambda i: (0, 0)),
        pl.BlockSpec((1, out), lambda i: (0, 0)),
    ]
    args = [acc2, wr, inp, dwT, db2]
    if hres is not None:
        in_specs.append(pl.BlockSpec((BN, out), lambda i: (i, 0)))
        args.append(hres)
    out_shape = [jax.ShapeDtypeStruct((Np, out), jnp.float32)]
    out_specs = [pl.BlockSpec((BN, out), lambda i: (i, 0))]
    if want_relu:
        out_shape.append(jax.ShapeDtypeStruct((Np, out), jnp.float32))
        out_specs.append(pl.BlockSpec((BN, out), lambda i: (i, 0)))
    res = pl.pallas_call(
        body,
        grid=grid,
        in_specs=in_specs,
        out_specs=out_specs,
        out_shape=out_shape,
    )(*args)
    return res if want_relu else (res[0],)


def _tc_layer0(acc2, wr, inp, dwT, db2):
    Np, K = acc2.shape
    oc = wr.shape[1]
    od = dwT.shape[1]
    inch = inp.shape[1]
    BN = 512
    grid = (Np // BN,)

    def body(acc_r, wr_r, inp_r, dw_r, db_r, h_r, r_r):
        c0 = jnp.dot(acc_r[...], wr_r[...], preferred_element_type=jnp.float32)
        d0 = jnp.dot(inp_r[...], dw_r[...],
                     preferred_element_type=jnp.float32) + db_r[...]
        h_r[:, :oc] = c0
        h_r[:, oc:] = d0
        r_r[:, :oc] = jnp.maximum(c0, 0.0)
        r_r[:, oc:] = jnp.maximum(d0, 0.0)

    return pl.pallas_call(
        body,
        grid=grid,
        in_specs=[
            pl.BlockSpec((BN, K), lambda i: (i, 0)),
            pl.BlockSpec((K, oc), lambda i: (0, 0)),
            pl.BlockSpec((BN, inch), lambda i: (i, 0)),
            pl.BlockSpec((inch, od), lambda i: (0, 0)),
            pl.BlockSpec((1, od), lambda i: (0, 0)),
        ],
        out_specs=[
            pl.BlockSpec((BN, oc + od), lambda i: (i, 0)),
            pl.BlockSpec((BN, oc + od), lambda i: (i, 0)),
        ],
        out_shape=[
            jax.ShapeDtypeStruct((Np, oc + od), jnp.float32),
            jax.ShapeDtypeStruct((Np, oc + od), jnp.float32),
        ],
    )(acc2, wr, inp, dwT, db2)


def _pad_rows(x, rows):
    return jnp.pad(x, ((0, rows - x.shape[0]), (0, 0)))


def _conv_w_flat(w, in_p):
    kk = w.shape[0] * w.shape[1] * w.shape[2]
    w = w.reshape(kk, w.shape[3], w.shape[4])
    if w.shape[1] != in_p:
        w = jnp.pad(w, ((0, 0), (0, in_p - w.shape[1]), (0, 0)))
    return w.reshape(kk * in_p, w.shape[2])


def kernel(xyz, feats, conv0_w, dense0_w, dense0_b, conv1_w, dense1_w,
           dense1_b, conv2_w, dense2_w, dense2_b, conv3_w, dense3_w,
           dense3_b, edge_src, edge_dst):
    N = xyz.shape[0]
    E = edge_src.shape[0]
    C = -(-N // _G)
    C = -(-C // _NW) * _NW
    Np = C * _G
    Ep = -(-(E + _SB) // 4096) * 4096

    xs = xyz[:, 0] + 0.0
    ys = xyz[:, 1] + 0.0
    zs = xyz[:, 2] + 0.0
    srcp = jnp.pad(edge_src, (0, Ep - E))
    dstp = jnp.pad(edge_dst, (0, Ep - E), constant_values=N - 1)
    offs = jnp.searchsorted(
        edge_dst, jnp.arange(C + 1, dtype=jnp.int32) * _G).astype(jnp.int32)

    wgt, rowb = _make_geometry_kernel(N, Ep)(xs, ys, zs, srcp, dstp)

    f0 = jnp.concatenate(
        [jnp.ones((N, 1), jnp.float32), feats,
         jnp.zeros((N, 3), jnp.float32)], axis=1)
    f0 = _pad_rows(f0, Np)

    acc0 = _make_scatter_kernel(Np, Ep, C, 16)(f0, wgt, rowb, offs, srcp)
    acc0 = acc0.reshape(Np, 64 * 16)
    w0r = _conv_w_flat(conv0_w, 16)
    dw0 = jnp.pad(dense0_w.T, ((0, 3), (0, 0)))
    h0, r0 = _tc_layer0(acc0, w0r, f0, dw0, dense0_b.reshape(1, -1))

    acc1 = _make_scatter_kernel(Np, Ep, C, 96)(r0, wgt, rowb, offs, srcp)
    acc1 = acc1.reshape(Np, 64 * 96)
    w1r = _conv_w_flat(conv1_w, 96)
    h1, r1 = _tc_layer(acc1, w1r, r0, dense1_w.T,
                       dense1_b.reshape(1, -1), None, True, 1.0)

    acc2 = _make_scatter_kernel(Np, Ep, C, 64)(r1, wgt, rowb, offs, srcp)
    acc2 = acc2.reshape(Np, 64 * 64)
    w2r = _conv_w_flat(conv2_w, 64)
    h2, r2 = _tc_layer(acc2, w2r, r1, dense2_w.T,
                       dense2_b.reshape(1, -1), h1, True, 1.0)

    acc3 = _make_scatter_kernel(Np, Ep, C, 64)(r2, wgt, rowb, offs, srcp)
    acc3 = acc3.reshape(Np, 64 * 64)
    w3r = jnp.pad(_conv_w_flat(conv3_w, 64), ((0, 0), (0, 5)))
    dw3 = jnp.pad(dense3_w.T, ((0, 0), (0, 5)))
    db3 = jnp.pad(dense3_b, (0, 5)).reshape(1, -1)
    (h3,) = _tc_layer(acc3, w3r, r2, dw3, db3, None, False, 1.0 / 128.0)

    return h3[:N, :3]


# speedup vs baseline: 9.9970x; 1.5102x over previous
import functools

import jax
import jax.numpy as jnp
import numpy as np
from jax import lax
from jax.experimental import pallas as pl
from jax.experimental.pallas import tpu as pltpu
from jax.experimental.pallas import tpu_sc as plsc

EXTENT = float(np.float32(1.5 * 6 * 0.025))
RADIUS = EXTENT / 2.0

_NC = 2
_NS = 16
_NW = _NC * _NS

_G = 16
_ROWS = _G * 64
_SB = 256
_GEO_B = 1024

_OFF = (0, 1, 4, 5, 16, 17, 20, 21)

_ATAN_C = (0.9999903820069088, -0.333001703445317, 0.19663803791358045,
           -0.12725099600657397, 0.07100970338731588, -0.026659183461222813,
           0.004672589475411147)


def _v_sqrt(v):
    i = lax.bitcast_convert_type(v, jnp.int32)
    y = lax.bitcast_convert_type(jnp.int32(0x5F3759DF) - (i >> 1), jnp.float32)
    for _ in range(3):
        y = y * (1.5 - 0.5 * v * y * y)
    return v * y


def _v_atan(x):
    x = jnp.clip(x, -1.1, 1.1)
    t = x * x
    p = jnp.float32(_ATAN_C[6])
    for k in range(5, -1, -1):
        p = p * t + jnp.float32(_ATAN_C[k])
    return p * x


def _sgn(v):
    return jnp.where(v >= 0.0, 1.0, -1.0)


def _edge_geometry(xs, ys, zs, xd, yd, zd):
    inv_r = jnp.float32(1.0 / RADIUS)
    x = (xs - xd) * inv_r
    y = (ys - yd) * inv_r
    z = (zs - zd) * inv_r
    eps = jnp.float32(1e-8)
    sq = x * x + y * y + z * z
    norm = _v_sqrt(sq + eps)
    condA = (5.0 / 4.0) * z * z > (x * x + y * y)
    sA = _v_sqrt(3.0 * norm / (norm + jnp.abs(z) + eps))
    rxy = _v_sqrt(x * x + y * y + eps)
    sB = norm / rxy
    cx = jnp.where(condA, x * sA, x * sB)
    cy = jnp.where(condA, y * sA, y * sB)
    cz = jnp.where(condA, _sgn(z) * norm, 1.5 * z)
    is0 = sq < 1e-12
    zero = jnp.zeros_like(cx)
    cx = jnp.where(is0, zero, cx)
    cy = jnp.where(is0, zero, cy)
    cz = jnp.where(is0, zero, cz)
    sq_xy = cx * cx + cy * cy
    r = _v_sqrt(sq_xy + eps)
    condC = jnp.abs(cy) <= jnp.abs(cx)
    denx = jnp.where(jnp.abs(cx) > eps, cx, jnp.float32(1.0))
    deny = jnp.where(jnp.abs(cy) > eps, cy, jnp.float32(1.0))
    aC = _sgn(cx) * r
    bC = aC * jnp.float32(4.0 / np.pi) * _v_atan(cy / denx)
    bD = _sgn(cy) * r
    aD = bD * jnp.float32(4.0 / np.pi) * _v_atan(cx / deny)
    ux = jnp.where(condC, aC, aD)
    uy = jnp.where(condC, bC, bD)
    xy0 = sq_xy < 1e-12
    ux = jnp.where(xy0, zero, ux)
    uy = jnp.where(xy0, zero, uy)

    one = jnp.float32(1.0)
    omsq = one - sq
    win = jnp.clip(omsq * omsq * omsq, 0.0, 1.0)
    mx = jnp.clip(ux, -1.0, 1.0)
    my = jnp.clip(uy, -1.0, 1.0)
    mz = jnp.clip(cz, -1.0, 1.0)
    tx = (mx + one) * 1.5
    ty = (my + one) * 1.5
    tz = (mz + one) * 1.5
    ix = jnp.minimum(tx.astype(jnp.int32), 2)
    iy = jnp.minimum(ty.astype(jnp.int32), 2)
    iz = jnp.minimum(tz.astype(jnp.int32), 2)
    wx1 = tx - ix.astype(jnp.float32)
    wy1 = ty - iy.astype(jnp.float32)
    wz1 = tz - iz.astype(jnp.float32)
    wx0 = one - wx1
    wy0 = one - wy1
    wz0 = one - wz1
    ws = []
    for wz in (wz0, wz1):
        for wy in (wy0, wy1):
            wyz = win * wz * wy
            for wx in (wx0, wx1):
                ws.append(wyz * wx)
    cellbase = (iz * 4 + iy) * 4 + ix
    return ws, cellbase


def _make_geometry_kernel(N, Ep):
    per_tile = Ep // _NW
    nbatch = (per_tile + _GEO_B - 1) // _GEO_B
    mesh = plsc.VectorSubcoreMesh(core_axis_name="c", subcore_axis_name="s")

    @functools.partial(
        pl.kernel,
        mesh=mesh,
        compiler_params=pltpu.CompilerParams(needs_layout_passes=False),
        out_type=jax.ShapeDtypeStruct((Ep * 16,), jnp.float32),
        scratch_types=[
            pltpu.VMEM((N,), jnp.float32),
            pltpu.VMEM((N,), jnp.float32),
            pltpu.VMEM((N,), jnp.float32),
            pltpu.VMEM((_GEO_B,), jnp.int32),
            pltpu.VMEM((_GEO_B,), jnp.int32),
            pltpu.VMEM((_GEO_B * 16,), jnp.float32),
        ],
    )
    def geom_kernel(xs_h, ys_h, zs_h, src_h, dst_h, rec_h,
                    xs_v, ys_v, zs_v, sbuf, dbuf, rout):
        wid = lax.axis_index("s") * _NC + lax.axis_index("c")
        base = wid * per_tile
        pltpu.sync_copy(xs_h, xs_v)
        pltpu.sync_copy(ys_h, ys_v)
        pltpu.sync_copy(zs_h, zs_v)
        iota = lax.iota(jnp.int32, 16)

        for b in range(nbatch):
            b0 = base + b * _GEO_B
            sb = min(_GEO_B, per_tile - b * _GEO_B)
            pltpu.sync_copy(src_h.at[pl.ds(b0, sb)], sbuf.at[pl.ds(0, sb)])
            pltpu.sync_copy(dst_h.at[pl.ds(b0, sb)], dbuf.at[pl.ds(0, sb)])

            @plsc.parallel_loop(0, sb // 16, unroll=2)
            def group(g):
                col = g * 16 + iota
                sv = plsc.load_gather(sbuf, [col])
                dv = plsc.load_gather(dbuf, [col])
                xsv = plsc.load_gather(xs_v, [sv])
                ysv = plsc.load_gather(ys_v, [sv])
                zsv = plsc.load_gather(zs_v, [sv])
                xdv = plsc.load_gather(xs_v, [dv])
                ydv = plsc.load_gather(ys_v, [dv])
                zdv = plsc.load_gather(zs_v, [dv])
                ws, cellbase = _edge_geometry(xsv, ysv, zsv, xdv, ydv, zdv)
                rec0 = col * 16
                rowf = plsc.bitcast(dv * 64 + cellbase, jnp.float32)
                plsc.store_scatter(rout, [rec0], rowf)
                for k in range(8):
                    plsc.store_scatter(rout, [rec0 + (k + 1)], ws[k])

            pltpu.sync_copy(rout.at[pl.ds(0, sb * 16)],
                            rec_h.at[pl.ds(b0 * 16, sb * 16)])

    return geom_kernel


def _make_scatter_kernel(Np, Ep, C, in_p):
    nq = in_p // 16
    chunks_per_tile = C // _NW
    acc_words = _ROWS * in_p
    mesh = plsc.VectorSubcoreMesh(core_axis_name="c", subcore_axis_name="s")

    @functools.partial(
        pl.kernel,
        mesh=mesh,
        compiler_params=pltpu.CompilerParams(needs_layout_passes=False,
                                             use_tc_tiling_on_sc=False),
        out_type=jax.ShapeDtypeStruct((Np * 64 * in_p,), jnp.float32),
        scratch_types=[
            pltpu.VMEM((acc_words,), jnp.float32),
            pltpu.VMEM((_SB, in_p), jnp.float32),
            pltpu.VMEM((_SB // 2,), jnp.int32),
            pltpu.VMEM((_SB // 2,), jnp.int32),
            pltpu.VMEM((_SB * 16,), jnp.float32),
            pltpu.VMEM((C + 1 + 16,), jnp.int32),
            pltpu.SemaphoreType.DMA,
        ],
    )
    def scatter_kernel(feat_h, rec_h, offs_h, src_h, acc_h,
                       accv, rows_v, sbuf0, sbuf1, wbuf, offs_v, sem):
        wid = lax.axis_index("s") * _NC + lax.axis_index("c")
        pltpu.sync_copy(offs_h, offs_v.at[pl.ds(0, C + 1)])
        iota = lax.iota(jnp.int32, 16)
        zeros16 = jnp.zeros((16,), jnp.float32)
        cols = [q * 16 + iota for q in range(nq)]

        def chunk_body(t, carry):
            c = wid + t * _NW
            ov = offs_v[pl.ds(c, 16)]
            e0 = ov[0]
            e1 = ov[1]
            row0 = c * _ROWS

            @plsc.parallel_loop(0, acc_words // 64, unroll=4)
            def zbody(i):
                b = i * 64
                for o in range(4):
                    plsc.store_scatter(accv, [b + o * 16 + iota], zeros16)

            b0a = pl.multiple_of(e0 & jnp.int32(~127), 128)
            nb = (e1 - b0a + jnp.int32(_SB - 1)) // jnp.int32(_SB)

            def batch_body(tb, carry3):
                b0 = pl.multiple_of(b0a + tb * _SB, 128)
                pltpu.sync_copy(src_h.at[pl.ds(b0, _SB // 2)], sbuf0)
                pltpu.sync_copy(src_h.at[pl.ds(b0 + _SB // 2, _SB // 2)],
                                sbuf1)
                pltpu.sync_copy(rec_h.at[pl.ds(b0 * 16, _SB * 16)], wbuf)
                cp0 = pltpu.async_copy(
                    feat_h.at[sbuf0], rows_v.at[pl.ds(0, _SB // 2)], sem)
                cp1 = pltpu.async_copy(
                    feat_h.at[sbuf1], rows_v.at[pl.ds(_SB // 2, _SB // 2)],
                    sem)
                js = jnp.maximum(e0 - b0, 0)
                je = jnp.minimum(e1 - b0, jnp.int32(_SB))
                half = jnp.int32(_SB // 2)
                js0 = js
                je0 = jnp.maximum(jnp.minimum(je, half), js0)
                js1 = jnp.maximum(js, half)
                je1 = jnp.maximum(je, js1)

                def edge_body(j):
                    rec = wbuf[pl.ds(pl.multiple_of(j * 16, 16), 16)]
                    reci = plsc.bitcast(rec, jnp.int32)
                    rowloc = reci[0] - row0
                    jv = lax.broadcast(j, (16,))
                    rv = [plsc.load_gather(rows_v, [jv, cols[q]])
                          for q in range(nq)]
                    for k in range(8):
                        wv = lax.broadcast(rec[k + 1], (16,))
                        rb = (rowloc + _OFF[k]) * in_p
                        for q in range(nq):
                            plsc.addupdate(
                                accv.at[pl.ds(rb + q * 16, 16)],
                                wv * rv[q])

                cp0.wait()
                plsc.parallel_loop(js0, je0, unroll=2)(edge_body)
                cp1.wait()
                plsc.parallel_loop(js1, je1, unroll=2)(edge_body)
                return carry3

            lax.fori_loop(0, nb, batch_body, jnp.int32(0))
            pltpu.sync_copy(
                accv,
                acc_h.at[pl.ds(pl.multiple_of(row0 * in_p, acc_words),
                               acc_words)])
            return carry

        lax.fori_loop(0, chunks_per_tile, chunk_body, jnp.int32(0))

    return scatter_kernel


def _tc_layer(acc2, wr, inp, dwT, db2, hres, want_relu, scale):
    Np, K = acc2.shape
    out = wr.shape[1]
    inch = inp.shape[1]
    BN = 512
    grid = (Np // BN,)

    def body(*refs):
        if hres is not None:
            acc_r, wr_r, inp_r, dw_r, db_r, hres_r = refs[:6]
            outs = refs[6:]
        else:
            acc_r, wr_r, inp_r, dw_r, db_r = refs[:5]
            outs = refs[5:]
        o = jnp.dot(acc_r[...], wr_r[...], preferred_element_type=jnp.float32)
        o = o + jnp.dot(inp_r[...], dw_r[...],
                        preferred_element_type=jnp.float32)
        o = o + db_r[...]
        if hres is not None:
            o = o + hres_r[...]
        if scale != 1.0:
            o = o * jnp.float32(scale)
        outs[0][...] = o
        if want_relu:
            outs[1][...] = jnp.maximum(o, 0.0)

    in_specs = [
        pl.BlockSpec((BN, K), lambda i: (i, 0)),
        pl.BlockSpec((K, out), lambda i: (0, 0)),
        pl.BlockSpec((BN, inch), lambda i: (i, 0)),
        pl.BlockSpec((inch, out), lambda i: (0, 0)),
        pl.BlockSpec((1, out), lambda i: (0, 0)),
    ]
    args = [acc2, wr, inp, dwT, db2]
    if hres is not None:
        in_specs.append(pl.BlockSpec((BN, out), lambda i: (i, 0)))
        args.append(hres)
    out_shape = [jax.ShapeDtypeStruct((Np, out), jnp.float32)]
    out_specs = [pl.BlockSpec((BN, out), lambda i: (i, 0))]
    if want_relu:
        out_shape.append(jax.ShapeDtypeStruct((Np, out), jnp.float32))
        out_specs.append(pl.BlockSpec((BN, out), lambda i: (i, 0)))
    res = pl.pallas_call(
        body,
        grid=grid,
        in_specs=in_specs,
        out_specs=out_specs,
        out_shape=out_shape,
    )(*args)
    return res if want_relu else (res[0],)


def _tc_layer0(acc2, wr, inp, dwT, db2):
    Np, K = acc2.shape
    oc = wr.shape[1]
    od = dwT.shape[1]
    inch = inp.shape[1]
    BN = 512
    grid = (Np // BN,)

    def body(acc_r, wr_r, inp_r, dw_r, db_r, h_r, r_r):
        c0 = jnp.dot(acc_r[...], wr_r[...], preferred_element_type=jnp.float32)
        d0 = jnp.dot(inp_r[...], dw_r[...],
                     preferred_element_type=jnp.float32) + db_r[...]
        h_r[:, :oc] = c0
        h_r[:, oc:] = d0
        r_r[:, :oc] = jnp.maximum(c0, 0.0)
        r_r[:, oc:] = jnp.maximum(d0, 0.0)

    return pl.pallas_call(
        body,
        grid=grid,
        in_specs=[
            pl.BlockSpec((BN, K), lambda i: (i, 0)),
            pl.BlockSpec((K, oc), lambda i: (0, 0)),
            pl.BlockSpec((BN, inch), lambda i: (i, 0)),
            pl.BlockSpec((inch, od), lambda i: (0, 0)),
            pl.BlockSpec((1, od), lambda i: (0, 0)),
        ],
        out_specs=[
            pl.BlockSpec((BN, oc + od), lambda i: (i, 0)),
            pl.BlockSpec((BN, oc + od), lambda i: (i, 0)),
        ],
        out_shape=[
            jax.ShapeDtypeStruct((Np, oc + od), jnp.float32),
            jax.ShapeDtypeStruct((Np, oc + od), jnp.float32),
        ],
    )(acc2, wr, inp, dwT, db2)


def _pad_rows(x, rows):
    return jnp.pad(x, ((0, rows - x.shape[0]), (0, 0)))


def _conv_w_flat(w, in_p):
    kk = w.shape[0] * w.shape[1] * w.shape[2]
    w = w.reshape(kk, w.shape[3], w.shape[4])
    if w.shape[1] != in_p:
        w = jnp.pad(w, ((0, 0), (0, in_p - w.shape[1]), (0, 0)))
    return w.reshape(kk * in_p, w.shape[2])


def kernel(xyz, feats, conv0_w, dense0_w, dense0_b, conv1_w, dense1_w,
           dense1_b, conv2_w, dense2_w, dense2_b, conv3_w, dense3_w,
           dense3_b, edge_src, edge_dst):
    N = xyz.shape[0]
    E = edge_src.shape[0]
    C = -(-N // _G)
    C = -(-C // _NW) * _NW
    Np = C * _G
    Ep = -(-(E + _SB) // 4096) * 4096

    xs = xyz[:, 0] + 0.0
    ys = xyz[:, 1] + 0.0
    zs = xyz[:, 2] + 0.0
    srcp = jnp.pad(edge_src, (0, Ep - E))
    dstp = jnp.pad(edge_dst, (0, Ep - E), constant_values=N - 1)
    offs = jnp.searchsorted(
        edge_dst, jnp.arange(C + 1, dtype=jnp.int32) * _G).astype(jnp.int32)

    rec = _make_geometry_kernel(N, Ep)(xs, ys, zs, srcp, dstp)

    f0 = jnp.concatenate(
        [jnp.ones((N, 1), jnp.float32), feats,
         jnp.zeros((N, 3), jnp.float32)], axis=1)
    f0 = _pad_rows(f0, Np)

    acc0 = _make_scatter_kernel(Np, Ep, C, 16)(f0, rec, offs, srcp)
    acc0 = acc0.reshape(Np, 64 * 16)
    w0r = _conv_w_flat(conv0_w, 16)
    dw0 = jnp.pad(dense0_w.T, ((0, 3), (0, 0)))
    h0, r0 = _tc_layer0(acc0, w0r, f0, dw0, dense0_b.reshape(1, -1))

    acc1 = _make_scatter_kernel(Np, Ep, C, 96)(r0, rec, offs, srcp)
    acc1 = acc1.reshape(Np, 64 * 96)
    w1r = _conv_w_flat(conv1_w, 96)
    h1, r1 = _tc_layer(acc1, w1r, r0, dense1_w.T,
                       dense1_b.reshape(1, -1), None, True, 1.0)

    acc2 = _make_scatter_kernel(Np, Ep, C, 64)(r1, rec, offs, srcp)
    acc2 = acc2.reshape(Np, 64 * 64)
    w2r = _conv_w_flat(conv2_w, 64)
    h2, r2 = _tc_layer(acc2, w2r, r1, dense2_w.T,
                       dense2_b.reshape(1, -1), h1, True, 1.0)

    acc3 = _make_scatter_kernel(Np, Ep, C, 64)(r2, rec, offs, srcp)
    acc3 = acc3.reshape(Np, 64 * 64)
    w3r = jnp.pad(_conv_w_flat(conv3_w, 64), ((0, 0), (0, 5)))
    dw3 = jnp.pad(dense3_w.T, ((0, 0), (0, 5)))
    db3 = jnp.pad(dense3_b, (0, 5)).reshape(1, -1)
    (h3,) = _tc_layer(acc3, w3r, r2, dw3, db3, None, False, 1.0 / 128.0)

    return h3[:N, :3]
